# baseline (speedup 1.0000x reference)
import jax
import jax.numpy as jnp
from jax import lax
from jax.experimental import pallas as pl
from jax.experimental.pallas import tpu as pltpu
from jax.experimental.pallas import tpu_sc as plsc

_B = 16384
_D = 64
_NW = 32
_CPW = 244
_IPW = _CPW * 128
_NCH = 122
_CIDS = 256
_TAIL0 = _NW * _IPW
_TPW = 72
_SLAB = 1664
_SLABW = _SLAB * _D
_LCAP = 896
_WCAP = 80
_DUMP = 2 * _B


def _vec16(x):
    return jnp.full((16,), x, jnp.int32)


def _popcnt(m):
    return jnp.max(plsc.all_reduce_population_count(m))


def _k1_body(uT, pT, uidx, pidx, utail, ptail,
             packed1d, positions1d, counts1d,
             buf, tailbuf, stage, ipiece, whid, wpos, wwid, pos_slab, cstage,
             sem0, sem1, sem2, sem3, sem4, sem5):
    wid = lax.axis_index("s") * 2 + lax.axis_index("c")
    lo = wid * _IPW
    col0 = wid * _CPW
    tlo = _TAIL0 + (wid - 24) * _TPW
    is_tail_w = wid >= 24
    toff = pl.multiple_of(jnp.maximum(0, (wid - 24) * _TPW), 8)
    iota = lax.iota(jnp.int32, 16)
    sems = (sem0, sem1, sem2, sem3, sem4, sem5)

    def pre_it(k, _):
        pos_slab[pl.ds(k * 16, 16)] = _DUMP + (iota & 7)
        return _
    lax.fori_loop(0, (_SLAB + 96) // 16, pre_it, 0)

    def chunk_copies(tbl, g, parity):
        cb = pl.multiple_of((col0 + 2 * g) * 128, 128)
        return [pltpu.make_async_copy(
            tbl.at[pl.ds(0, 64), pl.ds(cb + 128 * t, 128)],
            buf.at[parity, pl.ds(64 * t, 64), :],
            sems[parity]) for t in range(2)]

    def start_chunk(tbl, g, parity):
        for c in chunk_copies(tbl, g, parity):
            c.start()

    def wait_chunk(tbl, g, parity):
        for c in chunk_copies(tbl, g, parity):
            c.wait()

    def flush(cs_last):
        b = cs_last >> 6
        off = pl.multiple_of(wid * _SLABW + b * 4096, 8)
        pltpu.sync_copy(stage, packed1d.at[pl.ds(off, 4096)])

    cnt = jnp.int32(0)
    for tbl, tail, idx, par in ((uT, utail, uidx, 0), (pT, ptail, pidx, 1)):
        nl = jnp.int32(0)
        for p in range(4):
            pltpu.sync_copy(idx.at[pl.ds(p * 4096, 4096)], ipiece)

            def scan_it(k, nl):
                ids = ipiece[pl.ds(k * 16, 16)]
                m = (ids >= lo) & (ids < lo + _IPW)
                mt = is_tail_w & (ids >= tlo) & (ids < tlo + _TPW)
                m = m | mt
                j2 = 2 * (p * 4096 + k * 16 + iota) + par
                plsc.store_compressed(whid.at[pl.ds(nl, 16)], ids, mask=m)
                plsc.store_compressed(wpos.at[pl.ds(nl, 16)], j2, mask=m)
                return jnp.minimum(nl + _popcnt(m), _LCAP)
            nl = lax.fori_loop(0, 256, scan_it, nl)
        nvreg = (nl + 15) >> 4

        pltpu.sync_copy(tail.at[pl.ds(toff, _TPW), :], tailbuf)

        def process(g, parity, cnt):
            clo = lo + _CIDS * g
            wcap = jnp.minimum(_WCAP, _SLAB - cnt)

            def win_it(k, nw):
                ids = whid[pl.ds(k * 16, 16)]
                pos = wpos[pl.ds(k * 16, 16)]
                valid = (k * 16 + iota) < nl
                m = valid & (ids >= clo) & (ids < clo + _CIDS)
                plsc.store_compressed(wwid.at[pl.ds(nw, 16)], ids - clo, mask=m)
                plsc.store_compressed(pos_slab.at[pl.ds(cnt + nw, 16)], pos, mask=m)
                return jnp.minimum(nw + _popcnt(m), wcap)
            nw = lax.fori_loop(0, nvreg, win_it, jnp.int32(0))

            def ext_it(h, _):
                c = wwid[pl.ds(h, 16)][0]
                cs = cnt + h
                base = ((cs & 63) >> 1) * 128 + (cs & 1) * 64
                rbase = _vec16(64 * (c >> 7))
                cols = _vec16(c & 127)
                for k in range(4):
                    v = plsc.load_gather(
                        buf.at[parity], [rbase + iota + k * 16, cols])
                    stage[pl.ds(base + k * 16, 16)] = v

                @pl.when((cs & 63) == 63)
                def _fl():
                    flush(cs)
                return _
            lax.fori_loop(0, nw, ext_it, 0)
            return cnt + nw

        for r in range(6):
            start_chunk(tbl, r, r)

        def hex_it(q, cnt):
            for j in range(6):
                g = 6 * q + j
                wait_chunk(tbl, g, j)
                cnt = process(g, j, cnt)

                @pl.when(g + 6 < _NCH)
                def _nx():
                    start_chunk(tbl, g + 6, j)
            return cnt
        cnt = lax.fori_loop(0, _NCH // 6, hex_it, cnt)
        for g_last in (_NCH - 2, _NCH - 1):
            wait_chunk(tbl, g_last, g_last % 6)
            cnt = process(g_last, g_last % 6, cnt)

        wcap = jnp.minimum(_WCAP, _SLAB - cnt)

        def twin_it(k, nw):
            ids = whid[pl.ds(k * 16, 16)]
            pos = wpos[pl.ds(k * 16, 16)]
            valid = (k * 16 + iota) < nl
            m = valid & is_tail_w & (ids >= tlo) & (ids < tlo + _TPW)
            plsc.store_compressed(wwid.at[pl.ds(nw, 16)], ids - tlo, mask=m)
            plsc.store_compressed(pos_slab.at[pl.ds(cnt + nw, 16)], pos, mask=m)
            return jnp.minimum(nw + _popcnt(m), wcap)
        nwt = lax.fori_loop(0, nvreg, twin_it, jnp.int32(0))

        def text_it(h, _):
            c = wwid[pl.ds(h, 16)][0]
            cs = cnt + h
            base = ((cs & 63) >> 1) * 128 + (cs & 1) * 64
            for k in range(4):
                v = plsc.load_gather(tailbuf, [_vec16(c), iota + k * 16])
                stage[pl.ds(base + k * 16, 16)] = v

            @pl.when((cs & 63) == 63)
            def _fl():
                flush(cs)
            return _
        lax.fori_loop(0, nwt, text_it, 0)
        cnt = cnt + nwt

    @pl.when((cnt & 63) != 0)
    def _():
        flush(cnt)

    cstage[pl.ds(0, 16)] = _vec16(0) + cnt
    pltpu.sync_copy(cstage, counts1d.at[pl.ds(pl.multiple_of(wid * 16, 8), 16)])
    pltpu.sync_copy(pos_slab.at[pl.ds(0, _SLAB)],
                    positions1d.at[pl.ds(pl.multiple_of(wid * _SLAB, 8), _SLAB)])


def _k2_body(packed2, positions1d, counts1d, out32,
             pbuf, posv, idxbuf, cntv, sem):
    wid = lax.axis_index("s") * 2 + lax.axis_index("c")
    pltpu.sync_copy(counts1d.at[pl.ds(pl.multiple_of(wid * 16, 8), 16)], cntv)
    n = cntv[pl.ds(0, 16)][0]
    pltpu.sync_copy(positions1d.at[pl.ds(pl.multiple_of(wid * _SLAB, 8), _SLAB)], posv)

    def g_it(g, _):
        @pl.when(128 * g < n)
        def _inner():
            pltpu.sync_copy(
                packed2.at[pl.ds(pl.multiple_of(wid * _SLAB + 128 * g, 8), 128), :],
                pbuf)

            def cp_it(k, __):
                idxbuf[pl.ds(k * 16, 16)] = posv[pl.ds(128 * g + k * 16, 16)]
                return __
            lax.fori_loop(0, 8, cp_it, 0)
            pltpu.async_copy(pbuf, out32.at[idxbuf], sem).wait()
        return _
    lax.fori_loop(0, 13, g_it, 0)


def kernel(user, product, user_table, product_table):
    uidx = user.astype(jnp.int32)
    pidx = product.astype(jnp.int32)
    utail = jnp.pad(user_table[_TAIL0:], ((0, 0), (0, 64)))
    ptail = jnp.pad(product_table[_TAIL0:], ((0, 0), (0, 64)))
    mesh = plsc.VectorSubcoreMesh(core_axis_name="c", subcore_axis_name="s")

    k1 = pl.kernel(
        _k1_body,
        mesh=mesh,
        out_type=(
            jax.ShapeDtypeStruct((_NW * _SLABW,), jnp.float32),
            jax.ShapeDtypeStruct((_NW * _SLAB,), jnp.int32),
            jax.ShapeDtypeStruct((_NW * 16,), jnp.int32),
        ),
        scratch_types=[
            pltpu.VMEM((6, 128, 128), jnp.float32),
            pltpu.VMEM((_TPW, 128), jnp.float32),
            pltpu.VMEM((4096,), jnp.float32),
            pltpu.VMEM((4096,), jnp.int32),
            pltpu.VMEM((_LCAP + 16,), jnp.int32),
            pltpu.VMEM((_LCAP + 16,), jnp.int32),
            pltpu.VMEM((_WCAP + 16,), jnp.int32),
            pltpu.VMEM((_SLAB + 96,), jnp.int32),
            pltpu.VMEM((16,), jnp.int32),
            pltpu.SemaphoreType.DMA,
            pltpu.SemaphoreType.DMA,
            pltpu.SemaphoreType.DMA,
            pltpu.SemaphoreType.DMA,
            pltpu.SemaphoreType.DMA,
            pltpu.SemaphoreType.DMA,
        ],
        compiler_params=pltpu.CompilerParams(
            use_tc_tiling_on_sc=True, needs_layout_passes=False),
    )
    packed1d, positions1d, counts1d = k1(
        user_table.T, product_table.T, uidx, pidx, utail, ptail)

    packed2 = packed1d.reshape(_NW * _SLAB, _D)
    k2 = pl.kernel(
        _k2_body,
        mesh=mesh,
        out_type=jax.ShapeDtypeStruct((2 * _B + 8, _D), jnp.float32),
        scratch_types=[
            pltpu.VMEM((128, _D), jnp.float32),
            pltpu.VMEM((_SLAB,), jnp.int32),
            pltpu.VMEM((128,), jnp.int32),
            pltpu.VMEM((16,), jnp.int32),
            pltpu.SemaphoreType.DMA,
        ],
        compiler_params=pltpu.CompilerParams(use_tc_tiling_on_sc=False),
    )
    out32 = k2(packed2, positions1d, counts1d)
    return out32[:2 * _B].reshape(_B, 2 * _D)


# speedup vs baseline: 1.0897x; 1.0897x over previous
import jax
import jax.numpy as jnp
from jax import lax
from jax.experimental import pallas as pl
from jax.experimental.pallas import tpu as pltpu
from jax.experimental.pallas import tpu_sc as plsc

_B = 16384
_D = 64
_NW = 32
_CPW = 244
_IPW = _CPW * 128
_NCH = 61
_CIDS = 512
_TAIL0 = _NW * _IPW
_TPW = 72
_SLAB = 1664
_SLABW = _SLAB * _D
_LCAP = 896
_WCAP = 80
_DUMP = 2 * _B


def _vec16(x):
    return jnp.full((16,), x, jnp.int32)


def _popcnt(m):
    return jnp.max(plsc.all_reduce_population_count(m))


def _k1_body(uT, pT, uidx, pidx, utail, ptail,
             packed1d, positions1d, counts1d,
             buf, tailbuf, stage, ipiece, whid, wpos, wwid, pos_slab, cstage,
             colhit, sem0, sem1, sem2):
    wid = lax.axis_index("s") * 2 + lax.axis_index("c")
    lo = wid * _IPW
    col0 = wid * _CPW
    tlo = _TAIL0 + (wid - 24) * _TPW
    is_tail_w = wid >= 24
    toff = pl.multiple_of(jnp.maximum(0, (wid - 24) * _TPW), 8)
    iota = lax.iota(jnp.int32, 16)
    sems = (sem0, sem1, sem2)

    def pre_it(k, _):
        pos_slab[pl.ds(k * 16, 16)] = _DUMP + (iota & 7)
        return _
    lax.fori_loop(0, (_SLAB + 96) // 16, pre_it, 0)

    def chunk_copies(tbl, g, parity):
        cb = pl.multiple_of((col0 + 4 * g) * 128, 128)
        return [pltpu.make_async_copy(
            tbl.at[pl.ds(0, 64), pl.ds(cb + 128 * t, 128)],
            buf.at[parity, pl.ds(64 * t, 64), :],
            sems[parity]) for t in range(4)]

    def start_chunk(tbl, g, parity):
        hv = colhit[pl.ds(4 * g, 16)]
        for t, c in enumerate(chunk_copies(tbl, g, parity)):
            @pl.when(hv[t] != 0)
            def _st():
                c.start()

    def wait_chunk(tbl, g, parity):
        hv = colhit[pl.ds(4 * g, 16)]
        for t, c in enumerate(chunk_copies(tbl, g, parity)):
            @pl.when(hv[t] != 0)
            def _wt():
                c.wait()

    def flush(cs_last):
        b = cs_last >> 6
        off = pl.multiple_of(wid * _SLABW + b * 4096, 8)
        pltpu.sync_copy(stage, packed1d.at[pl.ds(off, 4096)])

    cnt = jnp.int32(0)
    for tbl, tail, idx, par in ((uT, utail, uidx, 0), (pT, ptail, pidx, 1)):
        def z_it(k, _):
            colhit[pl.ds(k * 16, 16)] = _vec16(0)
            return _
        lax.fori_loop(0, 17, z_it, 0)
        nl = jnp.int32(0)
        for p in range(4):
            pltpu.sync_copy(idx.at[pl.ds(p * 4096, 4096)], ipiece)

            def scan_it(k, nl):
                ids = ipiece[pl.ds(k * 16, 16)]
                mr = (ids >= lo) & (ids < lo + _IPW)
                mt = is_tail_w & (ids >= tlo) & (ids < tlo + _TPW)
                m = mr | mt
                j2 = 2 * (p * 4096 + k * 16 + iota) + par
                cl = jnp.clip((ids - lo) >> 7, 0, 259)
                plsc.store_scatter(colhit, [cl], _vec16(1), mask=mr)
                plsc.store_compressed(whid.at[pl.ds(nl, 16)], ids, mask=m)
                plsc.store_compressed(wpos.at[pl.ds(nl, 16)], j2, mask=m)
                return jnp.minimum(nl + _popcnt(m), _LCAP)
            nl = lax.fori_loop(0, 256, scan_it, nl)
        nvreg = (nl + 15) >> 4

        pltpu.sync_copy(tail.at[pl.ds(toff, _TPW), :], tailbuf)

        def process(g, parity, cnt):
            clo = lo + _CIDS * g
            wcap = jnp.minimum(_WCAP, _SLAB - cnt)

            def win_it(k, nw):
                ids = whid[pl.ds(k * 16, 16)]
                pos = wpos[pl.ds(k * 16, 16)]
                valid = (k * 16 + iota) < nl
                m = valid & (ids >= clo) & (ids < clo + _CIDS)
                plsc.store_compressed(wwid.at[pl.ds(nw, 16)], ids - clo, mask=m)
                plsc.store_compressed(pos_slab.at[pl.ds(cnt + nw, 16)], pos, mask=m)
                return jnp.minimum(nw + _popcnt(m), wcap)
            nw = lax.fori_loop(0, nvreg, win_it, jnp.int32(0))

            def ext_it(h, _):
                c = wwid[pl.ds(h, 16)][0]
                cs = cnt + h
                base = ((cs & 63) >> 1) * 128 + (cs & 1) * 64
                rbase = _vec16(64 * (c >> 7))
                cols = _vec16(c & 127)
                for k in range(4):
                    v = plsc.load_gather(
                        buf.at[parity], [rbase + iota + k * 16, cols])
                    stage[pl.ds(base + k * 16, 16)] = v

                @pl.when((cs & 63) == 63)
                def _fl():
                    flush(cs)
                return _
            lax.fori_loop(0, nw, ext_it, 0)
            return cnt + nw

        for r in range(3):
            start_chunk(tbl, r, r)

        def trip_it(q, cnt):
            for j in range(3):
                g = 3 * q + j
                wait_chunk(tbl, g, j)
                cnt = process(g, j, cnt)

                @pl.when(g + 3 < _NCH)
                def _nx():
                    start_chunk(tbl, g + 3, j)
            return cnt
        cnt = lax.fori_loop(0, _NCH // 3, trip_it, cnt)
        g_last = _NCH - 1
        wait_chunk(tbl, g_last, g_last % 3)
        cnt = process(g_last, g_last % 3, cnt)

        wcap = jnp.minimum(_WCAP, _SLAB - cnt)

        def twin_it(k, nw):
            ids = whid[pl.ds(k * 16, 16)]
            pos = wpos[pl.ds(k * 16, 16)]
            valid = (k * 16 + iota) < nl
            m = valid & is_tail_w & (ids >= tlo) & (ids < tlo + _TPW)
            plsc.store_compressed(wwid.at[pl.ds(nw, 16)], ids - tlo, mask=m)
            plsc.store_compressed(pos_slab.at[pl.ds(cnt + nw, 16)], pos, mask=m)
            return jnp.minimum(nw + _popcnt(m), wcap)
        nwt = lax.fori_loop(0, nvreg, twin_it, jnp.int32(0))

        def text_it(h, _):
            c = wwid[pl.ds(h, 16)][0]
            cs = cnt + h
            base = ((cs & 63) >> 1) * 128 + (cs & 1) * 64
            for k in range(4):
                v = plsc.load_gather(tailbuf, [_vec16(c), iota + k * 16])
                stage[pl.ds(base + k * 16, 16)] = v

            @pl.when((cs & 63) == 63)
            def _fl():
                flush(cs)
            return _
        lax.fori_loop(0, nwt, text_it, 0)
        cnt = cnt + nwt

    @pl.when((cnt & 63) != 0)
    def _():
        flush(cnt)

    cstage[pl.ds(0, 16)] = _vec16(0) + cnt
    pltpu.sync_copy(cstage, counts1d.at[pl.ds(pl.multiple_of(wid * 16, 8), 16)])
    pltpu.sync_copy(pos_slab.at[pl.ds(0, _SLAB)],
                    positions1d.at[pl.ds(pl.multiple_of(wid * _SLAB, 8), _SLAB)])


def _k2_body(packed2, positions1d, counts1d, out32,
             pbuf, posv, idxbuf, cntv, sem):
    wid = lax.axis_index("s") * 2 + lax.axis_index("c")
    pltpu.sync_copy(counts1d.at[pl.ds(pl.multiple_of(wid * 16, 8), 16)], cntv)
    n = cntv[pl.ds(0, 16)][0]
    pltpu.sync_copy(positions1d.at[pl.ds(pl.multiple_of(wid * _SLAB, 8), _SLAB)], posv)

    def g_it(g, _):
        @pl.when(128 * g < n)
        def _inner():
            pltpu.sync_copy(
                packed2.at[pl.ds(pl.multiple_of(wid * _SLAB + 128 * g, 8), 128), :],
                pbuf)

            def cp_it(k, __):
                idxbuf[pl.ds(k * 16, 16)] = posv[pl.ds(128 * g + k * 16, 16)]
                return __
            lax.fori_loop(0, 8, cp_it, 0)
            pltpu.async_copy(pbuf, out32.at[idxbuf], sem).wait()
        return _
    lax.fori_loop(0, 13, g_it, 0)


def kernel(user, product, user_table, product_table):
    uidx = user.astype(jnp.int32)
    pidx = product.astype(jnp.int32)
    utail = jnp.pad(user_table[_TAIL0:], ((0, 0), (0, 64)))
    ptail = jnp.pad(product_table[_TAIL0:], ((0, 0), (0, 64)))
    mesh = plsc.VectorSubcoreMesh(core_axis_name="c", subcore_axis_name="s")

    k1 = pl.kernel(
        _k1_body,
        mesh=mesh,
        out_type=(
            jax.ShapeDtypeStruct((_NW * _SLABW,), jnp.float32),
            jax.ShapeDtypeStruct((_NW * _SLAB,), jnp.int32),
            jax.ShapeDtypeStruct((_NW * 16,), jnp.int32),
        ),
        scratch_types=[
            pltpu.VMEM((3, 256, 128), jnp.float32),
            pltpu.VMEM((_TPW, 128), jnp.float32),
            pltpu.VMEM((4096,), jnp.float32),
            pltpu.VMEM((4096,), jnp.int32),
            pltpu.VMEM((_LCAP + 16,), jnp.int32),
            pltpu.VMEM((_LCAP + 16,), jnp.int32),
            pltpu.VMEM((_WCAP + 16,), jnp.int32),
            pltpu.VMEM((_SLAB + 96,), jnp.int32),
            pltpu.VMEM((16,), jnp.int32),
            pltpu.VMEM((272,), jnp.int32),
            pltpu.SemaphoreType.DMA,
            pltpu.SemaphoreType.DMA,
            pltpu.SemaphoreType.DMA,
        ],
        compiler_params=pltpu.CompilerParams(
            use_tc_tiling_on_sc=True, needs_layout_passes=False),
    )
    packed1d, positions1d, counts1d = k1(
        user_table.T, product_table.T, uidx, pidx, utail, ptail)

    packed2 = packed1d.reshape(_NW * _SLAB, _D)
    k2 = pl.kernel(
        _k2_body,
        mesh=mesh,
        out_type=jax.ShapeDtypeStruct((2 * _B + 8, _D), jnp.float32),
        scratch_types=[
            pltpu.VMEM((128, _D), jnp.float32),
            pltpu.VMEM((_SLAB,), jnp.int32),
            pltpu.VMEM((128,), jnp.int32),
            pltpu.VMEM((16,), jnp.int32),
            pltpu.SemaphoreType.DMA,
        ],
        compiler_params=pltpu.CompilerParams(use_tc_tiling_on_sc=False),
    )
    out32 = k2(packed2, positions1d, counts1d)
    return out32[:2 * _B].reshape(_B, 2 * _D)
